# baseline (speedup 1.0000x reference)
import functools

import jax
import jax.numpy as jnp
from jax import lax
from jax.experimental import pallas as pl
from jax.experimental.pallas import tpu as pltpu
from jax.experimental.pallas import tpu_sc as plsc

N = 10000
NP = 10112
E = 320000
ER = E // 128
ERP = 2560
EP = ERP * 128
F = 128
NC = 2
NS = 16
NW = NC * NS
RPW = ERP // NW
KB = 16
NB = RPW // KB
RPS = NP // NS

_mesh = plsc.VectorSubcoreMesh(core_axis_name="c", subcore_axis_name="s")
_sc_params = pltpu.CompilerParams(needs_layout_passes=False)


def _wid():
    return lax.axis_index("s") * NC + lax.axis_index("c")


def _deg_body(dst_hbm, z1k_hbm, degp_hbm, didx, ones_v, zv, deg_sp, sem):
    cid = lax.axis_index("c")
    sid = lax.axis_index("s")
    wid = _wid()
    for i in range(8):
        ones_v[pl.ds(i * 16, 16)] = jnp.full((16,), 1.0, jnp.float32)

    pltpu.sync_copy(z1k_hbm, zv)
    pltpu.sync_copy(zv, deg_sp.at[pl.ds(sid * RPS, RPS)])
    plsc.subcore_barrier()

    base = wid * RPW

    def bloop(b, c):
        pltpu.sync_copy(dst_hbm.at[pl.ds(base + b * KB, KB)], didx)
        cps = []
        for j in range(KB):
            cps.append(pltpu.async_copy(
                ones_v, deg_sp.at[didx.at[j]], sem, add=True))
        for cp in cps:
            cp.wait()
        return c

    lax.fori_loop(0, NB, bloop, 0)
    plsc.subcore_barrier()

    pltpu.sync_copy(deg_sp.at[pl.ds(sid * RPS, RPS)], zv)
    pltpu.sync_copy(zv, degp_hbm.at[pl.ds(cid * NP + sid * RPS, RPS)])


_deg_call = pl.kernel(
    _deg_body,
    out_type=jax.ShapeDtypeStruct((2 * NP,), jnp.float32),
    mesh=_mesh,
    compiler_params=_sc_params,
    scratch_types=[
        pltpu.VMEM((KB, 128), jnp.int32),
        pltpu.VMEM((128,), jnp.float32),
        pltpu.VMEM((RPS,), jnp.float32),
        pltpu.VMEM_SHARED((NP,), jnp.float32),
        pltpu.SemaphoreType.DMA,
    ],
)


def _agg1_body(src_hbm, dst_hbm, xs_hbm, zr_hbm, accp_hbm,
               sidx, didx, rows0, rows1, acc_sp, semg, sems):
    cid = lax.axis_index("c")
    sid = lax.axis_index("s")
    wid = _wid()

    pltpu.sync_copy(zr_hbm, rows0)
    for k in range(4):
        pltpu.sync_copy(rows0, acc_sp.at[pl.ds(sid * RPS + k * 128, 128)])
    pltpu.sync_copy(rows0.at[pl.ds(0, 120)],
                    acc_sp.at[pl.ds(sid * RPS + 512, 120)])
    plsc.subcore_barrier()

    base = wid * RPW
    bufs = (rows0, rows1)

    def bloop(b, c):
        off = base + b * KB
        pltpu.sync_copy(src_hbm.at[pl.ds(off, KB)], sidx)
        pltpu.sync_copy(dst_hbm.at[pl.ds(off, KB)], didx)
        gathers = [None] * KB
        scatters = [None] * KB
        gathers[0] = pltpu.async_copy(
            xs_hbm.at[sidx.at[0]], bufs[0], semg)
        for j in range(KB):
            if j >= 1:
                scatters[j - 1].wait()
            if j + 1 < KB:
                gathers[j + 1] = pltpu.async_copy(
                    xs_hbm.at[sidx.at[j + 1]], bufs[(j + 1) % 2], semg)
            gathers[j].wait()
            scatters[j] = pltpu.async_copy(
                bufs[j % 2], acc_sp.at[didx.at[j]], sems, add=True)
        scatters[KB - 1].wait()
        return c

    lax.fori_loop(0, NB, bloop, 0)
    plsc.subcore_barrier()

    for k in range(4):
        r = sid * RPS + k * 128
        pltpu.sync_copy(acc_sp.at[pl.ds(r, 128)], rows0)
        pltpu.sync_copy(rows0, accp_hbm.at[pl.ds(cid * NP + r, 128)])
    r = sid * RPS + 512
    pltpu.sync_copy(acc_sp.at[pl.ds(r, 120)], rows0.at[pl.ds(0, 120)])
    pltpu.sync_copy(rows0.at[pl.ds(0, 120)],
                    accp_hbm.at[pl.ds(cid * NP + r, 120)])


_agg1_call = pl.kernel(
    _agg1_body,
    out_type=jax.ShapeDtypeStruct((2 * NP, F), jnp.float32),
    mesh=_mesh,
    compiler_params=_sc_params,
    scratch_types=[
        pltpu.VMEM((KB, 128), jnp.int32),
        pltpu.VMEM((KB, 128), jnp.int32),
        pltpu.VMEM((128, F), jnp.float32),
        pltpu.VMEM((128, F), jnp.float32),
        pltpu.VMEM_SHARED((NP, F), jnp.float32),
        pltpu.SemaphoreType.DMA,
        pltpu.SemaphoreType.DMA,
    ],
)


def _agg2_body(src_hbm, dst_hbm, g0_hbm, g1_hbm, z1k_hbm,
               accA_hbm, accB_hbm,
               sidx, didx, g0v, g1v, v0b, v1b, zv, a0_sp, a1_sp, sem):
    cid = lax.axis_index("c")
    sid = lax.axis_index("s")
    wid = _wid()
    pltpu.sync_copy(g0_hbm, g0v)
    pltpu.sync_copy(g1_hbm, g1v)

    pltpu.sync_copy(z1k_hbm, zv)
    pltpu.sync_copy(zv, a0_sp.at[pl.ds(sid * RPS, RPS)])
    pltpu.sync_copy(zv, a1_sp.at[pl.ds(sid * RPS, RPS)])
    plsc.subcore_barrier()
    base = wid * RPW

    def bloop(b, c):
        off = base + b * KB
        pltpu.sync_copy(src_hbm.at[pl.ds(off, KB)], sidx)
        pltpu.sync_copy(dst_hbm.at[pl.ds(off, KB)], didx)
        cps = []
        for j in range(KB):
            jb = j % 2
            for k in range(8):
                sl = pl.ds(k * 16, 16)
                s = sidx[j, sl]
                v0b[jb, sl] = plsc.load_gather(g0v, [s])
                v1b[jb, sl] = plsc.load_gather(g1v, [s])
            if j >= 2:
                cps[2 * (j - 2)].wait()
                cps[2 * (j - 2) + 1].wait()
            cps.append(pltpu.async_copy(
                v0b.at[jb], a0_sp.at[didx.at[j]], sem, add=True))
            cps.append(pltpu.async_copy(
                v1b.at[jb], a1_sp.at[didx.at[j]], sem, add=True))
        for cp in cps[2 * (KB - 2):]:
            cp.wait()
        return c

    lax.fori_loop(0, NB, bloop, 0)
    plsc.subcore_barrier()

    pltpu.sync_copy(a0_sp.at[pl.ds(sid * RPS, RPS)], zv)
    pltpu.sync_copy(zv, accA_hbm.at[pl.ds(cid * NP + sid * RPS, RPS)])
    pltpu.sync_copy(a1_sp.at[pl.ds(sid * RPS, RPS)], zv)
    pltpu.sync_copy(zv, accB_hbm.at[pl.ds(cid * NP + sid * RPS, RPS)])


_agg2_call = pl.kernel(
    _agg2_body,
    out_type=(jax.ShapeDtypeStruct((2 * NP,), jnp.float32),
              jax.ShapeDtypeStruct((2 * NP,), jnp.float32)),
    mesh=_mesh,
    compiler_params=_sc_params,
    scratch_types=[
        pltpu.VMEM((KB, 128), jnp.int32),
        pltpu.VMEM((KB, 128), jnp.int32),
        pltpu.VMEM((NP,), jnp.float32),
        pltpu.VMEM((NP,), jnp.float32),
        pltpu.VMEM((2, 128), jnp.float32),
        pltpu.VMEM((2, 128), jnp.float32),
        pltpu.VMEM((RPS,), jnp.float32),
        pltpu.VMEM_SHARED((NP,), jnp.float32),
        pltpu.VMEM_SHARED((NP,), jnp.float32),
        pltpu.SemaphoreType.DMA,
    ],
)


def _head_body(src_hbm, dst_hbm, a0_hbm, a1_hbm, a2_hbm, a3_hbm,
               accA_hbm, accB_hbm, g0_hbm, g1_hbm, dv_hbm,
               tab_hbm, parb_hbm, o0_hbm, o1_hbm,
               p0v, p1v, aA0v, aA1v, aB0v, aB1v, g0v, g1v, dvv,
               tab_v, par_v, sidx, didx, a0v, a1v, a2v, a3v,
               o0_v, o1_v, sem):
    wid = _wid()
    pltpu.sync_copy(accA_hbm.at[pl.ds(0, NP)], aA0v)
    pltpu.sync_copy(accA_hbm.at[pl.ds(NP, NP)], aA1v)
    pltpu.sync_copy(accB_hbm.at[pl.ds(0, NP)], aB0v)
    pltpu.sync_copy(accB_hbm.at[pl.ds(NP, NP)], aB1v)
    pltpu.sync_copy(g0_hbm, g0v)
    pltpu.sync_copy(g1_hbm, g1v)
    pltpu.sync_copy(dv_hbm, dvv)
    pltpu.sync_copy(tab_hbm, tab_v)
    pltpu.sync_copy(parb_hbm, par_v)

    def ploop(i, c):
        sl = pl.ds(i * 16, 16)
        dv = dvv[sl]
        p0v[sl] = dv * (aA0v[sl] + aA1v[sl] + g0v[sl])
        p1v[sl] = dv * (aB0v[sl] + aB1v[sl] + g1v[sl])
        return c

    lax.fori_loop(0, NP // 16, ploop, 0)
    w00 = par_v[pl.ds(0, 16)]
    w01 = par_v[pl.ds(16, 16)]
    w10 = par_v[pl.ds(32, 16)]
    w11 = par_v[pl.ds(48, 16)]
    bo0 = par_v[pl.ds(64, 16)]
    bo1 = par_v[pl.ds(80, 16)]
    base = wid * RPW

    def bloop(b, c):
        off = base + b * KB
        cps = [pltpu.async_copy(src_hbm.at[pl.ds(off, KB)], sidx, sem),
               pltpu.async_copy(dst_hbm.at[pl.ds(off, KB)], didx, sem),
               pltpu.async_copy(a0_hbm.at[pl.ds(off, KB)], a0v, sem),
               pltpu.async_copy(a1_hbm.at[pl.ds(off, KB)], a1v, sem),
               pltpu.async_copy(a2_hbm.at[pl.ds(off, KB)], a2v, sem),
               pltpu.async_copy(a3_hbm.at[pl.ds(off, KB)], a3v, sem)]
        for cp in cps:
            cp.wait()
        for j in range(KB):
            for k in range(8):
                sl = pl.ds(k * 16, 16)
                s = sidx[j, sl]
                t = didx[j, sl]
                ps0 = plsc.load_gather(p0v, [s])
                ps1 = plsc.load_gather(p1v, [s])
                pt0 = plsc.load_gather(p0v, [t])
                pt1 = plsc.load_gather(p1v, [t])
                a0x = a0v[j, sl] * 8
                a1x = a1v[j, sl] * 8
                a2x = a2v[j, sl] * 8
                a3x = a3v[j, sl] * 8
                tA0 = plsc.load_gather(tab_v, [a0x])
                tA1 = plsc.load_gather(tab_v, [a0x + 1])
                tB0 = plsc.load_gather(tab_v, [a1x + 2])
                tB1 = plsc.load_gather(tab_v, [a1x + 3])
                tC0 = plsc.load_gather(tab_v, [a2x + 4])
                tC1 = plsc.load_gather(tab_v, [a2x + 5])
                tD0 = plsc.load_gather(tab_v, [a3x + 6])
                tD1 = plsc.load_gather(tab_v, [a3x + 7])
                z0 = jnp.maximum(ps0 - pt0 + tA0 + tB0 + tC0 + tD0, 0.0)
                z1 = jnp.maximum(ps1 - pt1 + tA1 + tB1 + tC1 + tD1, 0.0)
                o0_v[j, sl] = z0 * w00 + z1 * w10 + bo0
                o1_v[j, sl] = z0 * w01 + z1 * w11 + bo1
        pltpu.sync_copy(o0_v, o0_hbm.at[pl.ds(off, KB)])
        pltpu.sync_copy(o1_v, o1_hbm.at[pl.ds(off, KB)])
        return c

    lax.fori_loop(0, NB, bloop, 0)


_head_call = pl.kernel(
    _head_body,
    out_type=(jax.ShapeDtypeStruct((ERP, 128), jnp.float32),
              jax.ShapeDtypeStruct((ERP, 128), jnp.float32)),
    mesh=_mesh,
    compiler_params=_sc_params,
    scratch_types=[
        pltpu.VMEM((NP,), jnp.float32),
        pltpu.VMEM((NP,), jnp.float32),
        pltpu.VMEM((NP,), jnp.float32),
        pltpu.VMEM((NP,), jnp.float32),
        pltpu.VMEM((NP,), jnp.float32),
        pltpu.VMEM((NP,), jnp.float32),
        pltpu.VMEM((NP,), jnp.float32),
        pltpu.VMEM((NP,), jnp.float32),
        pltpu.VMEM((NP,), jnp.float32),
        pltpu.VMEM((256,), jnp.float32),
        pltpu.VMEM((96,), jnp.float32),
        pltpu.VMEM((KB, 128), jnp.int32),
        pltpu.VMEM((KB, 128), jnp.int32),
        pltpu.VMEM((KB, 128), jnp.int32),
        pltpu.VMEM((KB, 128), jnp.int32),
        pltpu.VMEM((KB, 128), jnp.int32),
        pltpu.VMEM((KB, 128), jnp.int32),
        pltpu.VMEM((KB, 128), jnp.float32),
        pltpu.VMEM((KB, 128), jnp.float32),
        pltpu.SemaphoreType.DMA,
    ],
)


_BR = 1000


def _xs_body(x_ref, w1_ref, d0_ref, d1_ref, xs_ref):
    dinv = lax.rsqrt(d0_ref[...] + d1_ref[...] + 1.0)
    xw = jnp.dot(x_ref[...], w1_ref[...], preferred_element_type=jnp.float32)
    xs_ref[...] = xw * dinv


def _xs_call(x, w1, d0, d1):
    return pl.pallas_call(
        _xs_body,
        grid=(N // _BR,),
        in_specs=[
            pl.BlockSpec((_BR, F), lambda i: (i, 0)),
            pl.BlockSpec((F, F), lambda i: (0, 0)),
            pl.BlockSpec((_BR, 1), lambda i: (i, 0)),
            pl.BlockSpec((_BR, 1), lambda i: (i, 0)),
        ],
        out_specs=pl.BlockSpec((_BR, F), lambda i: (i, 0)),
        out_shape=jax.ShapeDtypeStruct((NP, F), jnp.float32),
    )(x, w1, d0, d1)


def _mid_body(accp_ref, xs_ref, d0_ref, d1_ref, b1_ref, w2_ref, wfa_ref,
              wv_ref, emb0p_ref, emb1p_ref, wc_ref, wd_ref, bf1_ref,
              wf2_ref, bf2_ref, g0_ref, g1_ref, dv_ref, tab_ref, parb_ref):
    dinv = lax.rsqrt(d0_ref[...] + d1_ref[...] + 1.0)
    h1 = jnp.maximum(
        dinv * (accp_ref[0] + accp_ref[1] + xs_ref[...]) + b1_ref[...],
        0.0)
    m = jnp.dot(w2_ref[...], wfa_ref[...], preferred_element_type=jnp.float32)
    g = jnp.dot(h1, m, preferred_element_type=jnp.float32)
    gs = g * dinv
    g0_ref[...] = gs[:, 0:1]
    g1_ref[...] = gs[:, 1:2]
    dv_ref[...] = dinv
    iota_c = lax.broadcasted_iota(jnp.int32, (32, 1), 0).astype(jnp.float32)
    colA0 = iota_c * wv_ref[0, 0]
    colA1 = iota_c * wv_ref[0, 1]
    colB0 = iota_c * wv_ref[1, 0]
    colB1 = iota_c * wv_ref[1, 1]
    colC0 = jnp.dot(emb0p_ref[...], wc_ref[:, 0:1],
                    preferred_element_type=jnp.float32) + bf1_ref[0]
    colC1 = jnp.dot(emb0p_ref[...], wc_ref[:, 1:2],
                    preferred_element_type=jnp.float32) + bf1_ref[1]
    colD0 = jnp.dot(emb1p_ref[...], wd_ref[:, 0:1],
                    preferred_element_type=jnp.float32)
    colD1 = jnp.dot(emb1p_ref[...], wd_ref[:, 1:2],
                    preferred_element_type=jnp.float32)
    tab_ref[...] = jnp.concatenate(
        [colA0, colA1, colB0, colB1, colC0, colC1, colD0, colD1], axis=1)
    ones16 = jnp.ones((1, 16), jnp.float32)
    parb_ref[...] = jnp.concatenate([
        wf2_ref[0, 0] * ones16, wf2_ref[0, 1] * ones16,
        wf2_ref[1, 0] * ones16, wf2_ref[1, 1] * ones16,
        bf2_ref[0] * ones16, bf2_ref[1] * ones16,
    ], axis=0)


def _mid_call(accp, xs, d0, d1, b1, w2, wfa, wv, emb0p, emb1p, wc, wd,
              bf1, wf2, bf2):
    return pl.pallas_call(
        _mid_body,
        grid=(N // _BR,),
        in_specs=[
            pl.BlockSpec((2, _BR, F), lambda i: (0, i, 0)),
            pl.BlockSpec((_BR, F), lambda i: (i, 0)),
            pl.BlockSpec((_BR, 1), lambda i: (i, 0)),
            pl.BlockSpec((_BR, 1), lambda i: (i, 0)),
            pl.BlockSpec((F,), lambda i: (0,)),
            pl.BlockSpec((F, F), lambda i: (0, 0)),
            pl.BlockSpec((F, 2), lambda i: (0, 0)),
            pl.BlockSpec((2, 2), lambda i: (0, 0)),
            pl.BlockSpec((32, 32), lambda i: (0, 0)),
            pl.BlockSpec((32, 32), lambda i: (0, 0)),
            pl.BlockSpec((32, 2), lambda i: (0, 0)),
            pl.BlockSpec((32, 2), lambda i: (0, 0)),
            pl.BlockSpec((2,), lambda i: (0,)),
            pl.BlockSpec((2, 2), lambda i: (0, 0)),
            pl.BlockSpec((2,), lambda i: (0,)),
        ],
        out_specs=[
            pl.BlockSpec((_BR, 1), lambda i: (i, 0)),
            pl.BlockSpec((_BR, 1), lambda i: (i, 0)),
            pl.BlockSpec((_BR, 1), lambda i: (i, 0)),
            pl.BlockSpec((32, 8), lambda i: (0, 0)),
            pl.BlockSpec((6, 16), lambda i: (0, 0)),
        ],
        out_shape=[
            jax.ShapeDtypeStruct((NP, 1), jnp.float32),
            jax.ShapeDtypeStruct((NP, 1), jnp.float32),
            jax.ShapeDtypeStruct((NP, 1), jnp.float32),
            jax.ShapeDtypeStruct((32, 8), jnp.float32),
            jax.ShapeDtypeStruct((6, 16), jnp.float32),
        ],
    )(accp, xs, d0, d1, b1, w2, wfa, wv, emb0p, emb1p, wc, wd, bf1, wf2, bf2)


def _lsm_body(o0_ref, o1_ref, l0_ref, l1_ref):
    o0 = o0_ref[...]
    o1 = o1_ref[...]
    m = jnp.maximum(o0, o1)
    ls = m + jnp.log(jnp.exp(o0 - m) + jnp.exp(o1 - m))
    l0_ref[...] = o0 - ls
    l1_ref[...] = o1 - ls


def _lsm_call(o0r, o1r):
    spec = pl.BlockSpec((512, 128), lambda i: (i, 0))
    return pl.pallas_call(
        _lsm_body,
        grid=(ERP // 512,),
        in_specs=[spec, spec],
        out_specs=[spec, spec],
        out_shape=[
            jax.ShapeDtypeStruct((ERP, 128), jnp.float32),
            jax.ShapeDtypeStruct((ERP, 128), jnp.float32),
        ],
    )(o0r, o1r)


def kernel(x, edge_index, edge_attr, W1, b1, W2, b2, emb0, emb1,
           Wf1, bf1, Wf2, bf2):
    src = edge_index[0].astype(jnp.int32)
    dst = edge_index[1].astype(jnp.int32)
    ea = edge_attr.astype(jnp.int32)
    pad_idx = 10000 + (jnp.arange(EP - E, dtype=jnp.int32) % 8)
    pad_z = jnp.zeros((EP - E,), jnp.int32)

    def padr(a, v):
        return jnp.concatenate([a, v]).reshape(ERP, 128)

    srcP = padr(src, pad_idx)
    dstP = padr(dst, pad_idx)
    ea0P = padr(ea[0], pad_z)
    ea1P = padr(ea[1], pad_z)
    ea2P = padr(ea[2], pad_z)
    ea3P = padr(ea[3], pad_z)
    z1k = jnp.zeros((RPS,), jnp.float32)
    zrF = jnp.zeros((128, F), jnp.float32)
    emb0p = jnp.pad(emb0, ((0, 12), (0, 0)))
    emb1p = jnp.pad(emb1, ((0, 12), (0, 0)))
    wfa = Wf1[0:F]
    wv = Wf1[F:F + 2]
    wc = Wf1[F + 2:F + 34]
    wd = Wf1[F + 34:F + 66]

    degp = _deg_call(dstP, z1k)
    d0 = degp[0:N].reshape(N, 1)
    d1 = degp[NP:NP + N].reshape(N, 1)
    xsP = _xs_call(x, W1, d0, d1)
    accp = _agg1_call(srcP, dstP, xsP, zrF)
    g0, g1, dv, tab, parb = _mid_call(accp.reshape(2, NP, F),
                                      xsP.reshape(NP, F), d0, d1, b1, W2,
                                      wfa, wv, emb0p, emb1p,
                                      wc, wd, bf1, Wf2, bf2)
    g0P = g0.reshape(NP)
    g1P = g1.reshape(NP)
    accA, accB = _agg2_call(srcP, dstP, g0P, g1P, z1k)
    o0, o1 = _head_call(srcP, dstP, ea0P, ea1P, ea2P, ea3P,
                        accA, accB, g0P, g1P, dv.reshape(NP),
                        tab.reshape(256), parb.reshape(96))
    l0, l1 = _lsm_call(o0, o1)
    return jnp.concatenate(
        [l0.reshape(EP, 1)[0:E], l1.reshape(EP, 1)[0:E]], axis=1)


# speedup vs baseline: 1.0248x; 1.0248x over previous
import functools

import jax
import jax.numpy as jnp
from jax import lax
from jax.experimental import pallas as pl
from jax.experimental.pallas import tpu as pltpu
from jax.experimental.pallas import tpu_sc as plsc

N = 10000
NP = 10112
E = 320000
ER = E // 128
ERP = 2560
EP = ERP * 128
F = 128
NC = 2
NS = 16
NW = NC * NS
RPW = ERP // NW
KB = 16
NB = RPW // KB
RPS = NP // NS

_mesh = plsc.VectorSubcoreMesh(core_axis_name="c", subcore_axis_name="s")
_sc_params = pltpu.CompilerParams(needs_layout_passes=False)


def _wid():
    return lax.axis_index("s") * NC + lax.axis_index("c")


def _deg_body(dst_hbm, z1k_hbm, degp_hbm, didx, ones_v, zv, deg_sp, sem):
    cid = lax.axis_index("c")
    sid = lax.axis_index("s")
    wid = _wid()
    for i in range(8):
        ones_v[pl.ds(i * 16, 16)] = jnp.full((16,), 1.0, jnp.float32)

    pltpu.sync_copy(z1k_hbm, zv)
    pltpu.sync_copy(zv, deg_sp.at[pl.ds(sid * RPS, RPS)])
    plsc.subcore_barrier()

    base = wid * RPW

    def bloop(b, c):
        pltpu.sync_copy(dst_hbm.at[pl.ds(base + b * KB, KB)], didx)
        cps = []
        for j in range(KB):
            cps.append(pltpu.async_copy(
                ones_v, deg_sp.at[didx.at[j]], sem, add=True))
        for cp in cps:
            cp.wait()
        return c

    lax.fori_loop(0, NB, bloop, 0)
    plsc.subcore_barrier()

    pltpu.sync_copy(deg_sp.at[pl.ds(sid * RPS, RPS)], zv)
    pltpu.sync_copy(zv, degp_hbm.at[pl.ds(cid * NP + sid * RPS, RPS)])


_deg_call = pl.kernel(
    _deg_body,
    out_type=jax.ShapeDtypeStruct((2 * NP,), jnp.float32),
    mesh=_mesh,
    compiler_params=_sc_params,
    scratch_types=[
        pltpu.VMEM((KB, 128), jnp.int32),
        pltpu.VMEM((128,), jnp.float32),
        pltpu.VMEM((RPS,), jnp.float32),
        pltpu.VMEM_SHARED((NP,), jnp.float32),
        pltpu.SemaphoreType.DMA,
    ],
)


def _agg1_body(src_hbm, dst_hbm, xs_hbm, zr_hbm, accp_hbm,
               sidx, didx, rows0, rows1, acc_sp, semg, sems):
    cid = lax.axis_index("c")
    sid = lax.axis_index("s")
    wid = _wid()

    pltpu.sync_copy(zr_hbm, rows0)
    for k in range(4):
        pltpu.sync_copy(rows0, acc_sp.at[pl.ds(sid * RPS + k * 128, 128)])
    pltpu.sync_copy(rows0.at[pl.ds(0, 120)],
                    acc_sp.at[pl.ds(sid * RPS + 512, 120)])
    plsc.subcore_barrier()

    base = wid * RPW
    bufs = (rows0, rows1)

    def bloop(b, c):
        off = base + b * KB
        pltpu.sync_copy(src_hbm.at[pl.ds(off, KB)], sidx)
        pltpu.sync_copy(dst_hbm.at[pl.ds(off, KB)], didx)
        gathers = [None] * KB
        scatters = [None] * KB
        gathers[0] = pltpu.async_copy(
            xs_hbm.at[sidx.at[0]], bufs[0], semg)
        for j in range(KB):
            if j >= 1:
                scatters[j - 1].wait()
            if j + 1 < KB:
                gathers[j + 1] = pltpu.async_copy(
                    xs_hbm.at[sidx.at[j + 1]], bufs[(j + 1) % 2], semg)
            gathers[j].wait()
            scatters[j] = pltpu.async_copy(
                bufs[j % 2], acc_sp.at[didx.at[j]], sems, add=True)
        scatters[KB - 1].wait()
        return c

    lax.fori_loop(0, NB, bloop, 0)
    plsc.subcore_barrier()

    for k in range(4):
        r = sid * RPS + k * 128
        pltpu.sync_copy(acc_sp.at[pl.ds(r, 128)], rows0)
        pltpu.sync_copy(rows0, accp_hbm.at[pl.ds(cid * NP + r, 128)])
    r = sid * RPS + 512
    pltpu.sync_copy(acc_sp.at[pl.ds(r, 120)], rows0.at[pl.ds(0, 120)])
    pltpu.sync_copy(rows0.at[pl.ds(0, 120)],
                    accp_hbm.at[pl.ds(cid * NP + r, 120)])


_agg1_call = pl.kernel(
    _agg1_body,
    out_type=jax.ShapeDtypeStruct((2 * NP, F), jnp.float32),
    mesh=_mesh,
    compiler_params=_sc_params,
    scratch_types=[
        pltpu.VMEM((KB, 128), jnp.int32),
        pltpu.VMEM((KB, 128), jnp.int32),
        pltpu.VMEM((128, F), jnp.float32),
        pltpu.VMEM((128, F), jnp.float32),
        pltpu.VMEM_SHARED((NP, F), jnp.float32),
        pltpu.SemaphoreType.DMA,
        pltpu.SemaphoreType.DMA,
    ],
)


def _agg2_body(src_hbm, dst_hbm, g0_hbm, g1_hbm, z1k_hbm,
               accA_hbm, accB_hbm,
               sidx, didx, g0v, g1v, v0b, v1b, zv, g_sp, a0_sp, a1_sp, sem):
    cid = lax.axis_index("c")
    sid = lax.axis_index("s")
    wid = _wid()

    @pl.when(sid == 0)
    def _stage():
        pltpu.sync_copy(g0_hbm, g0v)
        pltpu.sync_copy(g1_hbm, g1v)
        pltpu.sync_copy(g0v, g_sp.at[pl.ds(0, NP)])
        pltpu.sync_copy(g1v, g_sp.at[pl.ds(NP, NP)])

    pltpu.sync_copy(z1k_hbm, zv)
    pltpu.sync_copy(zv, a0_sp.at[pl.ds(sid * RPS, RPS)])
    pltpu.sync_copy(zv, a1_sp.at[pl.ds(sid * RPS, RPS)])
    plsc.subcore_barrier()

    @pl.when(sid != 0)
    def _fetch():
        pltpu.sync_copy(g_sp.at[pl.ds(0, NP)], g0v)
        pltpu.sync_copy(g_sp.at[pl.ds(NP, NP)], g1v)
    base = wid * RPW

    def bloop(b, c):
        off = base + b * KB
        pltpu.sync_copy(src_hbm.at[pl.ds(off, KB)], sidx)
        pltpu.sync_copy(dst_hbm.at[pl.ds(off, KB)], didx)
        cps = []
        for j in range(KB):
            jb = j % 2
            for k in range(8):
                sl = pl.ds(k * 16, 16)
                s = sidx[j, sl]
                v0b[jb, sl] = plsc.load_gather(g0v, [s])
                v1b[jb, sl] = plsc.load_gather(g1v, [s])
            if j >= 2:
                cps[2 * (j - 2)].wait()
                cps[2 * (j - 2) + 1].wait()
            cps.append(pltpu.async_copy(
                v0b.at[jb], a0_sp.at[didx.at[j]], sem, add=True))
            cps.append(pltpu.async_copy(
                v1b.at[jb], a1_sp.at[didx.at[j]], sem, add=True))
        for cp in cps[2 * (KB - 2):]:
            cp.wait()
        return c

    lax.fori_loop(0, NB, bloop, 0)
    plsc.subcore_barrier()

    pltpu.sync_copy(a0_sp.at[pl.ds(sid * RPS, RPS)], zv)
    pltpu.sync_copy(zv, accA_hbm.at[pl.ds(cid * NP + sid * RPS, RPS)])
    pltpu.sync_copy(a1_sp.at[pl.ds(sid * RPS, RPS)], zv)
    pltpu.sync_copy(zv, accB_hbm.at[pl.ds(cid * NP + sid * RPS, RPS)])


_agg2_call = pl.kernel(
    _agg2_body,
    out_type=(jax.ShapeDtypeStruct((2 * NP,), jnp.float32),
              jax.ShapeDtypeStruct((2 * NP,), jnp.float32)),
    mesh=_mesh,
    compiler_params=_sc_params,
    scratch_types=[
        pltpu.VMEM((KB, 128), jnp.int32),
        pltpu.VMEM((KB, 128), jnp.int32),
        pltpu.VMEM((NP,), jnp.float32),
        pltpu.VMEM((NP,), jnp.float32),
        pltpu.VMEM((2, 128), jnp.float32),
        pltpu.VMEM((2, 128), jnp.float32),
        pltpu.VMEM((RPS,), jnp.float32),
        pltpu.VMEM_SHARED((2 * NP,), jnp.float32),
        pltpu.VMEM_SHARED((NP,), jnp.float32),
        pltpu.VMEM_SHARED((NP,), jnp.float32),
        pltpu.SemaphoreType.DMA,
    ],
)


def _head_body(src_hbm, dst_hbm, a0_hbm, a1_hbm, a2_hbm, a3_hbm,
               accA_hbm, accB_hbm, g0_hbm, g1_hbm, dv_hbm,
               tab_hbm, parb_hbm, o0_hbm, o1_hbm,
               p0v, p1v, aA0v, aA1v, aB0v, aB1v, g0v, g1v, dvv,
               tab_v, par_v, sidx, didx, a0v, a1v, a2v, a3v,
               o0_v, o1_v, p_sp, sem):
    sid = lax.axis_index("s")
    wid = _wid()
    pltpu.sync_copy(tab_hbm, tab_v)
    pltpu.sync_copy(parb_hbm, par_v)

    @pl.when(sid == 0)
    def _makep():
        pltpu.sync_copy(accA_hbm.at[pl.ds(0, NP)], aA0v)
        pltpu.sync_copy(accA_hbm.at[pl.ds(NP, NP)], aA1v)
        pltpu.sync_copy(accB_hbm.at[pl.ds(0, NP)], aB0v)
        pltpu.sync_copy(accB_hbm.at[pl.ds(NP, NP)], aB1v)
        pltpu.sync_copy(g0_hbm, g0v)
        pltpu.sync_copy(g1_hbm, g1v)
        pltpu.sync_copy(dv_hbm, dvv)

        def ploop(i, c):
            sl = pl.ds(i * 16, 16)
            dv = dvv[sl]
            p0v[sl] = dv * (aA0v[sl] + aA1v[sl] + g0v[sl])
            p1v[sl] = dv * (aB0v[sl] + aB1v[sl] + g1v[sl])
            return c

        lax.fori_loop(0, NP // 16, ploop, 0)
        pltpu.sync_copy(p0v, p_sp.at[pl.ds(0, NP)])
        pltpu.sync_copy(p1v, p_sp.at[pl.ds(NP, NP)])

    plsc.subcore_barrier()

    @pl.when(sid != 0)
    def _fetchp():
        pltpu.sync_copy(p_sp.at[pl.ds(0, NP)], p0v)
        pltpu.sync_copy(p_sp.at[pl.ds(NP, NP)], p1v)
    w00 = par_v[pl.ds(0, 16)]
    w01 = par_v[pl.ds(16, 16)]
    w10 = par_v[pl.ds(32, 16)]
    w11 = par_v[pl.ds(48, 16)]
    bo0 = par_v[pl.ds(64, 16)]
    bo1 = par_v[pl.ds(80, 16)]
    base = wid * RPW

    def bloop(b, c):
        off = base + b * KB
        cps = [pltpu.async_copy(src_hbm.at[pl.ds(off, KB)], sidx, sem),
               pltpu.async_copy(dst_hbm.at[pl.ds(off, KB)], didx, sem),
               pltpu.async_copy(a0_hbm.at[pl.ds(off, KB)], a0v, sem),
               pltpu.async_copy(a1_hbm.at[pl.ds(off, KB)], a1v, sem),
               pltpu.async_copy(a2_hbm.at[pl.ds(off, KB)], a2v, sem),
               pltpu.async_copy(a3_hbm.at[pl.ds(off, KB)], a3v, sem)]
        for cp in cps:
            cp.wait()
        for j in range(KB):
            for k in range(8):
                sl = pl.ds(k * 16, 16)
                s = sidx[j, sl]
                t = didx[j, sl]
                ps0 = plsc.load_gather(p0v, [s])
                ps1 = plsc.load_gather(p1v, [s])
                pt0 = plsc.load_gather(p0v, [t])
                pt1 = plsc.load_gather(p1v, [t])
                a0x = a0v[j, sl] * 8
                a1x = a1v[j, sl] * 8
                a2x = a2v[j, sl] * 8
                a3x = a3v[j, sl] * 8
                tA0 = plsc.load_gather(tab_v, [a0x])
                tA1 = plsc.load_gather(tab_v, [a0x + 1])
                tB0 = plsc.load_gather(tab_v, [a1x + 2])
                tB1 = plsc.load_gather(tab_v, [a1x + 3])
                tC0 = plsc.load_gather(tab_v, [a2x + 4])
                tC1 = plsc.load_gather(tab_v, [a2x + 5])
                tD0 = plsc.load_gather(tab_v, [a3x + 6])
                tD1 = plsc.load_gather(tab_v, [a3x + 7])
                z0 = jnp.maximum(ps0 - pt0 + tA0 + tB0 + tC0 + tD0, 0.0)
                z1 = jnp.maximum(ps1 - pt1 + tA1 + tB1 + tC1 + tD1, 0.0)
                o0_v[j, sl] = z0 * w00 + z1 * w10 + bo0
                o1_v[j, sl] = z0 * w01 + z1 * w11 + bo1
        pltpu.sync_copy(o0_v, o0_hbm.at[pl.ds(off, KB)])
        pltpu.sync_copy(o1_v, o1_hbm.at[pl.ds(off, KB)])
        return c

    lax.fori_loop(0, NB, bloop, 0)


_head_call = pl.kernel(
    _head_body,
    out_type=(jax.ShapeDtypeStruct((ERP, 128), jnp.float32),
              jax.ShapeDtypeStruct((ERP, 128), jnp.float32)),
    mesh=_mesh,
    compiler_params=_sc_params,
    scratch_types=[
        pltpu.VMEM((NP,), jnp.float32),
        pltpu.VMEM((NP,), jnp.float32),
        pltpu.VMEM((NP,), jnp.float32),
        pltpu.VMEM((NP,), jnp.float32),
        pltpu.VMEM((NP,), jnp.float32),
        pltpu.VMEM((NP,), jnp.float32),
        pltpu.VMEM((NP,), jnp.float32),
        pltpu.VMEM((NP,), jnp.float32),
        pltpu.VMEM((NP,), jnp.float32),
        pltpu.VMEM((256,), jnp.float32),
        pltpu.VMEM((96,), jnp.float32),
        pltpu.VMEM((KB, 128), jnp.int32),
        pltpu.VMEM((KB, 128), jnp.int32),
        pltpu.VMEM((KB, 128), jnp.int32),
        pltpu.VMEM((KB, 128), jnp.int32),
        pltpu.VMEM((KB, 128), jnp.int32),
        pltpu.VMEM((KB, 128), jnp.int32),
        pltpu.VMEM((KB, 128), jnp.float32),
        pltpu.VMEM((KB, 128), jnp.float32),
        pltpu.VMEM_SHARED((2 * NP,), jnp.float32),
        pltpu.SemaphoreType.DMA,
    ],
)


_BR = 1000


def _xs_body(x_ref, w1_ref, d0_ref, d1_ref, xs_ref):
    dinv = lax.rsqrt(d0_ref[...] + d1_ref[...] + 1.0)
    xw = jnp.dot(x_ref[...], w1_ref[...], preferred_element_type=jnp.float32)
    xs_ref[...] = xw * dinv


def _xs_call(x, w1, d0, d1):
    return pl.pallas_call(
        _xs_body,
        grid=(N // _BR,),
        in_specs=[
            pl.BlockSpec((_BR, F), lambda i: (i, 0)),
            pl.BlockSpec((F, F), lambda i: (0, 0)),
            pl.BlockSpec((_BR, 1), lambda i: (i, 0)),
            pl.BlockSpec((_BR, 1), lambda i: (i, 0)),
        ],
        out_specs=pl.BlockSpec((_BR, F), lambda i: (i, 0)),
        out_shape=jax.ShapeDtypeStruct((NP, F), jnp.float32),
    )(x, w1, d0, d1)


def _mid_body(accp_ref, xs_ref, d0_ref, d1_ref, b1_ref, w2_ref, wfa_ref,
              wv_ref, emb0p_ref, emb1p_ref, wc_ref, wd_ref, bf1_ref,
              wf2_ref, bf2_ref, g0_ref, g1_ref, dv_ref, tab_ref, parb_ref):
    dinv = lax.rsqrt(d0_ref[...] + d1_ref[...] + 1.0)
    h1 = jnp.maximum(
        dinv * (accp_ref[0] + accp_ref[1] + xs_ref[...]) + b1_ref[...],
        0.0)
    m = jnp.dot(w2_ref[...], wfa_ref[...], preferred_element_type=jnp.float32)
    g = jnp.dot(h1, m, preferred_element_type=jnp.float32)
    gs = g * dinv
    g0_ref[...] = gs[:, 0:1]
    g1_ref[...] = gs[:, 1:2]
    dv_ref[...] = dinv
    iota_c = lax.broadcasted_iota(jnp.int32, (32, 1), 0).astype(jnp.float32)
    colA0 = iota_c * wv_ref[0, 0]
    colA1 = iota_c * wv_ref[0, 1]
    colB0 = iota_c * wv_ref[1, 0]
    colB1 = iota_c * wv_ref[1, 1]
    colC0 = jnp.dot(emb0p_ref[...], wc_ref[:, 0:1],
                    preferred_element_type=jnp.float32) + bf1_ref[0]
    colC1 = jnp.dot(emb0p_ref[...], wc_ref[:, 1:2],
                    preferred_element_type=jnp.float32) + bf1_ref[1]
    colD0 = jnp.dot(emb1p_ref[...], wd_ref[:, 0:1],
                    preferred_element_type=jnp.float32)
    colD1 = jnp.dot(emb1p_ref[...], wd_ref[:, 1:2],
                    preferred_element_type=jnp.float32)
    tab_ref[...] = jnp.concatenate(
        [colA0, colA1, colB0, colB1, colC0, colC1, colD0, colD1], axis=1)
    ones16 = jnp.ones((1, 16), jnp.float32)
    parb_ref[...] = jnp.concatenate([
        wf2_ref[0, 0] * ones16, wf2_ref[0, 1] * ones16,
        wf2_ref[1, 0] * ones16, wf2_ref[1, 1] * ones16,
        bf2_ref[0] * ones16, bf2_ref[1] * ones16,
    ], axis=0)


def _mid_call(accp, xs, d0, d1, b1, w2, wfa, wv, emb0p, emb1p, wc, wd,
              bf1, wf2, bf2):
    return pl.pallas_call(
        _mid_body,
        grid=(N // _BR,),
        in_specs=[
            pl.BlockSpec((2, _BR, F), lambda i: (0, i, 0)),
            pl.BlockSpec((_BR, F), lambda i: (i, 0)),
            pl.BlockSpec((_BR, 1), lambda i: (i, 0)),
            pl.BlockSpec((_BR, 1), lambda i: (i, 0)),
            pl.BlockSpec((F,), lambda i: (0,)),
            pl.BlockSpec((F, F), lambda i: (0, 0)),
            pl.BlockSpec((F, 2), lambda i: (0, 0)),
            pl.BlockSpec((2, 2), lambda i: (0, 0)),
            pl.BlockSpec((32, 32), lambda i: (0, 0)),
            pl.BlockSpec((32, 32), lambda i: (0, 0)),
            pl.BlockSpec((32, 2), lambda i: (0, 0)),
            pl.BlockSpec((32, 2), lambda i: (0, 0)),
            pl.BlockSpec((2,), lambda i: (0,)),
            pl.BlockSpec((2, 2), lambda i: (0, 0)),
            pl.BlockSpec((2,), lambda i: (0,)),
        ],
        out_specs=[
            pl.BlockSpec((_BR, 1), lambda i: (i, 0)),
            pl.BlockSpec((_BR, 1), lambda i: (i, 0)),
            pl.BlockSpec((_BR, 1), lambda i: (i, 0)),
            pl.BlockSpec((32, 8), lambda i: (0, 0)),
            pl.BlockSpec((6, 16), lambda i: (0, 0)),
        ],
        out_shape=[
            jax.ShapeDtypeStruct((NP, 1), jnp.float32),
            jax.ShapeDtypeStruct((NP, 1), jnp.float32),
            jax.ShapeDtypeStruct((NP, 1), jnp.float32),
            jax.ShapeDtypeStruct((32, 8), jnp.float32),
            jax.ShapeDtypeStruct((6, 16), jnp.float32),
        ],
    )(accp, xs, d0, d1, b1, w2, wfa, wv, emb0p, emb1p, wc, wd, bf1, wf2, bf2)


def _lsm_body(o0_ref, o1_ref, l0_ref, l1_ref):
    o0 = o0_ref[...]
    o1 = o1_ref[...]
    m = jnp.maximum(o0, o1)
    ls = m + jnp.log(jnp.exp(o0 - m) + jnp.exp(o1 - m))
    l0_ref[...] = o0 - ls
    l1_ref[...] = o1 - ls


def _lsm_call(o0r, o1r):
    spec = pl.BlockSpec((ER, 128), lambda: (0, 0))
    return pl.pallas_call(
        _lsm_body,
        in_specs=[spec, spec],
        out_specs=[spec, spec],
        out_shape=[
            jax.ShapeDtypeStruct((ER, 128), jnp.float32),
            jax.ShapeDtypeStruct((ER, 128), jnp.float32),
        ],
    )(o0r, o1r)


def kernel(x, edge_index, edge_attr, W1, b1, W2, b2, emb0, emb1,
           Wf1, bf1, Wf2, bf2):
    src = edge_index[0].astype(jnp.int32)
    dst = edge_index[1].astype(jnp.int32)
    ea = edge_attr.astype(jnp.int32)
    pad_idx = 10000 + (jnp.arange(EP - E, dtype=jnp.int32) % 8)
    pad_z = jnp.zeros((EP - E,), jnp.int32)

    def padr(a, v):
        return jnp.concatenate([a, v]).reshape(ERP, 128)

    srcP = padr(src, pad_idx)
    dstP = padr(dst, pad_idx)
    ea0P = padr(ea[0], pad_z)
    ea1P = padr(ea[1], pad_z)
    ea2P = padr(ea[2], pad_z)
    ea3P = padr(ea[3], pad_z)
    z1k = jnp.zeros((RPS,), jnp.float32)
    zrF = jnp.zeros((128, F), jnp.float32)
    emb0p = jnp.pad(emb0, ((0, 12), (0, 0)))
    emb1p = jnp.pad(emb1, ((0, 12), (0, 0)))
    wfa = Wf1[0:F]
    wv = Wf1[F:F + 2]
    wc = Wf1[F + 2:F + 34]
    wd = Wf1[F + 34:F + 66]

    degp = _deg_call(dstP, z1k)
    d0 = degp[0:N].reshape(N, 1)
    d1 = degp[NP:NP + N].reshape(N, 1)
    xsP = _xs_call(x, W1, d0, d1)
    accp = _agg1_call(srcP, dstP, xsP, zrF)
    g0, g1, dv, tab, parb = _mid_call(accp.reshape(2, NP, F),
                                      xsP.reshape(NP, F), d0, d1, b1, W2,
                                      wfa, wv, emb0p, emb1p,
                                      wc, wd, bf1, Wf2, bf2)
    g0P = g0.reshape(NP)
    g1P = g1.reshape(NP)
    accA, accB = _agg2_call(srcP, dstP, g0P, g1P, z1k)
    o0, o1 = _head_call(srcP, dstP, ea0P, ea1P, ea2P, ea3P,
                        accA, accB, g0P, g1P, dv.reshape(NP),
                        tab.reshape(256), parb.reshape(96))
    l0, l1 = _lsm_call(o0[0:ER], o1[0:ER])
    return jnp.concatenate(
        [l0.reshape(E, 1), l1.reshape(E, 1)], axis=1)


# speedup vs baseline: 1.0249x; 1.0001x over previous
import jax
import jax.numpy as jnp
from jax import lax
from jax.experimental import pallas as pl
from jax.experimental.pallas import tpu as pltpu
from jax.experimental.pallas import tpu_sc as plsc

N = 10000
NP = 10112
E = 320000
ER = E // 128
ERP = 2560
EP = ERP * 128
F = 128
NC = 2
NS = 16
NW = NC * NS
RPW = ERP // NW
KB = 16
NB = RPW // KB
RPS = NP // NS

_mesh = plsc.VectorSubcoreMesh(core_axis_name="c", subcore_axis_name="s")
_sc_params = pltpu.CompilerParams(needs_layout_passes=False)


def _wid():
    return lax.axis_index("s") * NC + lax.axis_index("c")


def _deg_body(dst_hbm, z1k_hbm, degp_hbm, didx, ones_v, zv, deg_sp, sem):
    cid = lax.axis_index("c")
    sid = lax.axis_index("s")
    wid = _wid()
    for i in range(8):
        ones_v[pl.ds(i * 16, 16)] = jnp.full((16,), 1.0, jnp.float32)

    pltpu.sync_copy(z1k_hbm, zv)
    pltpu.sync_copy(zv, deg_sp.at[pl.ds(sid * RPS, RPS)])
    plsc.subcore_barrier()

    base = wid * RPW

    def bloop(b, c):
        pltpu.sync_copy(dst_hbm.at[pl.ds(base + b * KB, KB)], didx)
        cps = []
        for j in range(KB):
            cps.append(pltpu.async_copy(
                ones_v, deg_sp.at[didx.at[j]], sem, add=True))
        for cp in cps:
            cp.wait()
        return c

    lax.fori_loop(0, NB, bloop, 0)
    plsc.subcore_barrier()

    pltpu.sync_copy(deg_sp.at[pl.ds(sid * RPS, RPS)], zv)
    pltpu.sync_copy(zv, degp_hbm.at[pl.ds(cid * NP + sid * RPS, RPS)])


_deg_call = pl.kernel(
    _deg_body,
    out_type=jax.ShapeDtypeStruct((2 * NP,), jnp.float32),
    mesh=_mesh,
    compiler_params=_sc_params,
    scratch_types=[
        pltpu.VMEM((KB, 128), jnp.int32),
        pltpu.VMEM((128,), jnp.float32),
        pltpu.VMEM((RPS,), jnp.float32),
        pltpu.VMEM_SHARED((NP,), jnp.float32),
        pltpu.SemaphoreType.DMA,
    ],
)


def _agg1_body(src_hbm, dst_hbm, xs_hbm, zr_hbm, accp_hbm,
               sidx, didx, rows0, rows1, acc_sp, semg, sems):
    cid = lax.axis_index("c")
    sid = lax.axis_index("s")
    wid = _wid()

    pltpu.sync_copy(zr_hbm, rows0)
    for k in range(4):
        pltpu.sync_copy(rows0, acc_sp.at[pl.ds(sid * RPS + k * 128, 128)])
    pltpu.sync_copy(rows0.at[pl.ds(0, 120)],
                    acc_sp.at[pl.ds(sid * RPS + 512, 120)])
    plsc.subcore_barrier()

    base = wid * RPW
    bufs = (rows0, rows1)

    def bloop(b, c):
        off = base + b * KB
        pltpu.sync_copy(src_hbm.at[pl.ds(off, KB)], sidx)
        pltpu.sync_copy(dst_hbm.at[pl.ds(off, KB)], didx)
        gathers = [None] * KB
        scatters = [None] * KB
        gathers[0] = pltpu.async_copy(
            xs_hbm.at[sidx.at[0]], bufs[0], semg)
        for j in range(KB):
            if j >= 1:
                scatters[j - 1].wait()
            if j + 1 < KB:
                gathers[j + 1] = pltpu.async_copy(
                    xs_hbm.at[sidx.at[j + 1]], bufs[(j + 1) % 2], semg)
            gathers[j].wait()
            scatters[j] = pltpu.async_copy(
                bufs[j % 2], acc_sp.at[didx.at[j]], sems, add=True)
        scatters[KB - 1].wait()
        return c

    lax.fori_loop(0, NB, bloop, 0)
    plsc.subcore_barrier()

    for k in range(4):
        r = sid * RPS + k * 128
        pltpu.sync_copy(acc_sp.at[pl.ds(r, 128)], rows0)
        pltpu.sync_copy(rows0, accp_hbm.at[pl.ds(cid * NP + r, 128)])
    r = sid * RPS + 512
    pltpu.sync_copy(acc_sp.at[pl.ds(r, 120)], rows0.at[pl.ds(0, 120)])
    pltpu.sync_copy(rows0.at[pl.ds(0, 120)],
                    accp_hbm.at[pl.ds(cid * NP + r, 120)])


_agg1_call = pl.kernel(
    _agg1_body,
    out_type=jax.ShapeDtypeStruct((2 * NP, F), jnp.float32),
    mesh=_mesh,
    compiler_params=_sc_params,
    scratch_types=[
        pltpu.VMEM((KB, 128), jnp.int32),
        pltpu.VMEM((KB, 128), jnp.int32),
        pltpu.VMEM((128, F), jnp.float32),
        pltpu.VMEM((128, F), jnp.float32),
        pltpu.VMEM_SHARED((NP, F), jnp.float32),
        pltpu.SemaphoreType.DMA,
        pltpu.SemaphoreType.DMA,
    ],
)


def _agg2_body(src_hbm, dst_hbm, g0_hbm, g1_hbm, z1k_hbm,
               accA_hbm, accB_hbm,
               sidx, didx, g0v, g1v, v0b, v1b, zv, g_sp, a0_sp, a1_sp, sem):
    cid = lax.axis_index("c")
    sid = lax.axis_index("s")
    wid = _wid()

    @pl.when(sid == 0)
    def _stage():
        pltpu.sync_copy(g0_hbm, g0v)
        pltpu.sync_copy(g1_hbm, g1v)
        pltpu.sync_copy(g0v, g_sp.at[pl.ds(0, NP)])
        pltpu.sync_copy(g1v, g_sp.at[pl.ds(NP, NP)])

    pltpu.sync_copy(z1k_hbm, zv)
    pltpu.sync_copy(zv, a0_sp.at[pl.ds(sid * RPS, RPS)])
    pltpu.sync_copy(zv, a1_sp.at[pl.ds(sid * RPS, RPS)])
    plsc.subcore_barrier()

    @pl.when(sid != 0)
    def _fetch():
        pltpu.sync_copy(g_sp.at[pl.ds(0, NP)], g0v)
        pltpu.sync_copy(g_sp.at[pl.ds(NP, NP)], g1v)
    base = wid * RPW

    def bloop(b, c):
        off = base + b * KB
        pltpu.sync_copy(src_hbm.at[pl.ds(off, KB)], sidx)
        pltpu.sync_copy(dst_hbm.at[pl.ds(off, KB)], didx)
        cps = []
        for j in range(KB):
            jb = j % 2
            for k in range(8):
                sl = pl.ds(k * 16, 16)
                s = sidx[j, sl]
                v0b[jb, sl] = plsc.load_gather(g0v, [s])
                v1b[jb, sl] = plsc.load_gather(g1v, [s])
            if j >= 2:
                cps[2 * (j - 2)].wait()
                cps[2 * (j - 2) + 1].wait()
            cps.append(pltpu.async_copy(
                v0b.at[jb], a0_sp.at[didx.at[j]], sem, add=True))
            cps.append(pltpu.async_copy(
                v1b.at[jb], a1_sp.at[didx.at[j]], sem, add=True))
        for cp in cps[2 * (KB - 2):]:
            cp.wait()
        return c

    lax.fori_loop(0, NB, bloop, 0)
    plsc.subcore_barrier()

    pltpu.sync_copy(a0_sp.at[pl.ds(sid * RPS, RPS)], zv)
    pltpu.sync_copy(zv, accA_hbm.at[pl.ds(cid * NP + sid * RPS, RPS)])
    pltpu.sync_copy(a1_sp.at[pl.ds(sid * RPS, RPS)], zv)
    pltpu.sync_copy(zv, accB_hbm.at[pl.ds(cid * NP + sid * RPS, RPS)])


_agg2_call = pl.kernel(
    _agg2_body,
    out_type=(jax.ShapeDtypeStruct((2 * NP,), jnp.float32),
              jax.ShapeDtypeStruct((2 * NP,), jnp.float32)),
    mesh=_mesh,
    compiler_params=_sc_params,
    scratch_types=[
        pltpu.VMEM((KB, 128), jnp.int32),
        pltpu.VMEM((KB, 128), jnp.int32),
        pltpu.VMEM((NP,), jnp.float32),
        pltpu.VMEM((NP,), jnp.float32),
        pltpu.VMEM((2, 128), jnp.float32),
        pltpu.VMEM((2, 128), jnp.float32),
        pltpu.VMEM((RPS,), jnp.float32),
        pltpu.VMEM_SHARED((2 * NP,), jnp.float32),
        pltpu.VMEM_SHARED((NP,), jnp.float32),
        pltpu.VMEM_SHARED((NP,), jnp.float32),
        pltpu.SemaphoreType.DMA,
    ],
)


def _head_body(src_hbm, dst_hbm, a0_hbm, a1_hbm, a2_hbm, a3_hbm,
               accA_hbm, accB_hbm, g0_hbm, g1_hbm, dv_hbm,
               tab_hbm, parb_hbm, o0_hbm, o1_hbm,
               p0v, p1v, aA0v, aA1v, aB0v, aB1v, g0v, g1v, dvv,
               tab_v, par_v, sidx, didx, a0v, a1v, a2v, a3v,
               o0_v, o1_v, p_sp, sem):
    sid = lax.axis_index("s")
    wid = _wid()
    pltpu.sync_copy(tab_hbm, tab_v)
    pltpu.sync_copy(parb_hbm, par_v)

    @pl.when(sid == 0)
    def _makep():
        pltpu.sync_copy(accA_hbm.at[pl.ds(0, NP)], aA0v)
        pltpu.sync_copy(accA_hbm.at[pl.ds(NP, NP)], aA1v)
        pltpu.sync_copy(accB_hbm.at[pl.ds(0, NP)], aB0v)
        pltpu.sync_copy(accB_hbm.at[pl.ds(NP, NP)], aB1v)
        pltpu.sync_copy(g0_hbm, g0v)
        pltpu.sync_copy(g1_hbm, g1v)
        pltpu.sync_copy(dv_hbm, dvv)

        def ploop(i, c):
            sl = pl.ds(i * 16, 16)
            dv = dvv[sl]
            p0v[sl] = dv * (aA0v[sl] + aA1v[sl] + g0v[sl])
            p1v[sl] = dv * (aB0v[sl] + aB1v[sl] + g1v[sl])
            return c

        lax.fori_loop(0, NP // 16, ploop, 0)
        pltpu.sync_copy(p0v, p_sp.at[pl.ds(0, NP)])
        pltpu.sync_copy(p1v, p_sp.at[pl.ds(NP, NP)])

    plsc.subcore_barrier()

    @pl.when(sid != 0)
    def _fetchp():
        pltpu.sync_copy(p_sp.at[pl.ds(0, NP)], p0v)
        pltpu.sync_copy(p_sp.at[pl.ds(NP, NP)], p1v)
    w00 = par_v[pl.ds(0, 16)]
    w01 = par_v[pl.ds(16, 16)]
    w10 = par_v[pl.ds(32, 16)]
    w11 = par_v[pl.ds(48, 16)]
    bo0 = par_v[pl.ds(64, 16)]
    bo1 = par_v[pl.ds(80, 16)]
    base = wid * RPW

    def bloop(b, c):
        off = base + b * KB
        cps = [pltpu.async_copy(src_hbm.at[pl.ds(off, KB)], sidx, sem),
               pltpu.async_copy(dst_hbm.at[pl.ds(off, KB)], didx, sem),
               pltpu.async_copy(a0_hbm.at[pl.ds(off, KB)], a0v, sem),
               pltpu.async_copy(a1_hbm.at[pl.ds(off, KB)], a1v, sem),
               pltpu.async_copy(a2_hbm.at[pl.ds(off, KB)], a2v, sem),
               pltpu.async_copy(a3_hbm.at[pl.ds(off, KB)], a3v, sem)]
        for cp in cps:
            cp.wait()
        for j in range(KB):
            for k in range(8):
                sl = pl.ds(k * 16, 16)
                s = sidx[j, sl]
                t = didx[j, sl]
                ps0 = plsc.load_gather(p0v, [s])
                ps1 = plsc.load_gather(p1v, [s])
                pt0 = plsc.load_gather(p0v, [t])
                pt1 = plsc.load_gather(p1v, [t])
                a0x = a0v[j, sl] * 8
                a1x = a1v[j, sl] * 8
                a2x = a2v[j, sl] * 8
                a3x = a3v[j, sl] * 8
                tA0 = plsc.load_gather(tab_v, [a0x])
                tA1 = plsc.load_gather(tab_v, [a0x + 1])
                tB0 = plsc.load_gather(tab_v, [a1x + 2])
                tB1 = plsc.load_gather(tab_v, [a1x + 3])
                tC0 = plsc.load_gather(tab_v, [a2x + 4])
                tC1 = plsc.load_gather(tab_v, [a2x + 5])
                tD0 = plsc.load_gather(tab_v, [a3x + 6])
                tD1 = plsc.load_gather(tab_v, [a3x + 7])
                z0 = jnp.maximum(ps0 - pt0 + tA0 + tB0 + tC0 + tD0, 0.0)
                z1 = jnp.maximum(ps1 - pt1 + tA1 + tB1 + tC1 + tD1, 0.0)
                o0_v[j, sl] = z0 * w00 + z1 * w10 + bo0
                o1_v[j, sl] = z0 * w01 + z1 * w11 + bo1
        pltpu.sync_copy(o0_v, o0_hbm.at[pl.ds(off, KB)])
        pltpu.sync_copy(o1_v, o1_hbm.at[pl.ds(off, KB)])
        return c

    lax.fori_loop(0, NB, bloop, 0)


_head_call = pl.kernel(
    _head_body,
    out_type=(jax.ShapeDtypeStruct((ERP, 128), jnp.float32),
              jax.ShapeDtypeStruct((ERP, 128), jnp.float32)),
    mesh=_mesh,
    compiler_params=_sc_params,
    scratch_types=[
        pltpu.VMEM((NP,), jnp.float32),
        pltpu.VMEM((NP,), jnp.float32),
        pltpu.VMEM((NP,), jnp.float32),
        pltpu.VMEM((NP,), jnp.float32),
        pltpu.VMEM((NP,), jnp.float32),
        pltpu.VMEM((NP,), jnp.float32),
        pltpu.VMEM((NP,), jnp.float32),
        pltpu.VMEM((NP,), jnp.float32),
        pltpu.VMEM((NP,), jnp.float32),
        pltpu.VMEM((256,), jnp.float32),
        pltpu.VMEM((96,), jnp.float32),
        pltpu.VMEM((KB, 128), jnp.int32),
        pltpu.VMEM((KB, 128), jnp.int32),
        pltpu.VMEM((KB, 128), jnp.int32),
        pltpu.VMEM((KB, 128), jnp.int32),
        pltpu.VMEM((KB, 128), jnp.int32),
        pltpu.VMEM((KB, 128), jnp.int32),
        pltpu.VMEM((KB, 128), jnp.float32),
        pltpu.VMEM((KB, 128), jnp.float32),
        pltpu.VMEM_SHARED((2 * NP,), jnp.float32),
        pltpu.SemaphoreType.DMA,
    ],
)


_BR = 1000


def _xs_body(x_ref, w1_ref, d0_ref, d1_ref, xs_ref):
    dinv = lax.rsqrt(d0_ref[...] + d1_ref[...] + 1.0)
    xw = jnp.dot(x_ref[...], w1_ref[...], preferred_element_type=jnp.float32)
    xs_ref[...] = xw * dinv


def _xs_call(x, w1, d0, d1):
    return pl.pallas_call(
        _xs_body,
        grid=(N // _BR,),
        in_specs=[
            pl.BlockSpec((_BR, F), lambda i: (i, 0)),
            pl.BlockSpec((F, F), lambda i: (0, 0)),
            pl.BlockSpec((_BR, 1), lambda i: (i, 0)),
            pl.BlockSpec((_BR, 1), lambda i: (i, 0)),
        ],
        out_specs=pl.BlockSpec((_BR, F), lambda i: (i, 0)),
        out_shape=jax.ShapeDtypeStruct((NP, F), jnp.float32),
    )(x, w1, d0, d1)


def _mid_body(accp_ref, xs_ref, d0_ref, d1_ref, b1_ref, w2_ref, wfa_ref,
              wv_ref, emb0p_ref, emb1p_ref, wc_ref, wd_ref, bf1_ref,
              wf2_ref, bf2_ref, g0_ref, g1_ref, dv_ref, tab_ref, parb_ref):
    dinv = lax.rsqrt(d0_ref[...] + d1_ref[...] + 1.0)
    h1 = jnp.maximum(
        dinv * (accp_ref[0] + accp_ref[1] + xs_ref[...]) + b1_ref[...],
        0.0)
    m = jnp.dot(w2_ref[...], wfa_ref[...], preferred_element_type=jnp.float32)
    g = jnp.dot(h1, m, preferred_element_type=jnp.float32)
    gs = g * dinv
    g0_ref[...] = gs[:, 0:1]
    g1_ref[...] = gs[:, 1:2]
    dv_ref[...] = dinv
    iota_c = lax.broadcasted_iota(jnp.int32, (32, 1), 0).astype(jnp.float32)
    colA0 = iota_c * wv_ref[0, 0]
    colA1 = iota_c * wv_ref[0, 1]
    colB0 = iota_c * wv_ref[1, 0]
    colB1 = iota_c * wv_ref[1, 1]
    colC0 = jnp.dot(emb0p_ref[...], wc_ref[:, 0:1],
                    preferred_element_type=jnp.float32) + bf1_ref[0]
    colC1 = jnp.dot(emb0p_ref[...], wc_ref[:, 1:2],
                    preferred_element_type=jnp.float32) + bf1_ref[1]
    colD0 = jnp.dot(emb1p_ref[...], wd_ref[:, 0:1],
                    preferred_element_type=jnp.float32)
    colD1 = jnp.dot(emb1p_ref[...], wd_ref[:, 1:2],
                    preferred_element_type=jnp.float32)
    tab_ref[...] = jnp.concatenate(
        [colA0, colA1, colB0, colB1, colC0, colC1, colD0, colD1], axis=1)
    ones16 = jnp.ones((1, 16), jnp.float32)
    parb_ref[...] = jnp.concatenate([
        wf2_ref[0, 0] * ones16, wf2_ref[0, 1] * ones16,
        wf2_ref[1, 0] * ones16, wf2_ref[1, 1] * ones16,
        bf2_ref[0] * ones16, bf2_ref[1] * ones16,
    ], axis=0)


def _mid_call(accp, xs, d0, d1, b1, w2, wfa, wv, emb0p, emb1p, wc, wd,
              bf1, wf2, bf2):
    return pl.pallas_call(
        _mid_body,
        grid=(N // _BR,),
        in_specs=[
            pl.BlockSpec((2, _BR, F), lambda i: (0, i, 0)),
            pl.BlockSpec((_BR, F), lambda i: (i, 0)),
            pl.BlockSpec((_BR, 1), lambda i: (i, 0)),
            pl.BlockSpec((_BR, 1), lambda i: (i, 0)),
            pl.BlockSpec((F,), lambda i: (0,)),
            pl.BlockSpec((F, F), lambda i: (0, 0)),
            pl.BlockSpec((F, 2), lambda i: (0, 0)),
            pl.BlockSpec((2, 2), lambda i: (0, 0)),
            pl.BlockSpec((32, 32), lambda i: (0, 0)),
            pl.BlockSpec((32, 32), lambda i: (0, 0)),
            pl.BlockSpec((32, 2), lambda i: (0, 0)),
            pl.BlockSpec((32, 2), lambda i: (0, 0)),
            pl.BlockSpec((2,), lambda i: (0,)),
            pl.BlockSpec((2, 2), lambda i: (0, 0)),
            pl.BlockSpec((2,), lambda i: (0,)),
        ],
        out_specs=[
            pl.BlockSpec((_BR, 1), lambda i: (i, 0)),
            pl.BlockSpec((_BR, 1), lambda i: (i, 0)),
            pl.BlockSpec((_BR, 1), lambda i: (i, 0)),
            pl.BlockSpec((32, 8), lambda i: (0, 0)),
            pl.BlockSpec((6, 16), lambda i: (0, 0)),
        ],
        out_shape=[
            jax.ShapeDtypeStruct((NP, 1), jnp.float32),
            jax.ShapeDtypeStruct((NP, 1), jnp.float32),
            jax.ShapeDtypeStruct((NP, 1), jnp.float32),
            jax.ShapeDtypeStruct((32, 8), jnp.float32),
            jax.ShapeDtypeStruct((6, 16), jnp.float32),
        ],
    )(accp, xs, d0, d1, b1, w2, wfa, wv, emb0p, emb1p, wc, wd, bf1, wf2, bf2)


def _lsm_body(o0_ref, o1_ref, l0_ref, l1_ref):
    o0 = o0_ref[...]
    o1 = o1_ref[...]
    m = jnp.maximum(o0, o1)
    ls = m + jnp.log(jnp.exp(o0 - m) + jnp.exp(o1 - m))
    l0_ref[...] = o0 - ls
    l1_ref[...] = o1 - ls


def _lsm_call(o0r, o1r):
    spec = pl.BlockSpec((ER, 128), lambda: (0, 0))
    return pl.pallas_call(
        _lsm_body,
        in_specs=[spec, spec],
        out_specs=[spec, spec],
        out_shape=[
            jax.ShapeDtypeStruct((ER, 128), jnp.float32),
            jax.ShapeDtypeStruct((ER, 128), jnp.float32),
        ],
    )(o0r, o1r)


def kernel(x, edge_index, edge_attr, W1, b1, W2, b2, emb0, emb1,
           Wf1, bf1, Wf2, bf2):
    src = edge_index[0].astype(jnp.int32)
    dst = edge_index[1].astype(jnp.int32)
    ea = edge_attr.astype(jnp.int32)
    pad_idx = 10000 + (jnp.arange(EP - E, dtype=jnp.int32) % 8)
    pad_z = jnp.zeros((EP - E,), jnp.int32)

    def padr(a, v):
        return jnp.concatenate([a, v]).reshape(ERP, 128)

    srcP = padr(src, pad_idx)
    dstP = padr(dst, pad_idx)
    ea0P = padr(ea[0], pad_z)
    ea1P = padr(ea[1], pad_z)
    ea2P = padr(ea[2], pad_z)
    ea3P = padr(ea[3], pad_z)
    z1k = jnp.zeros((RPS,), jnp.float32)
    zrF = jnp.zeros((128, F), jnp.float32)
    emb0p = jnp.pad(emb0, ((0, 12), (0, 0)))
    emb1p = jnp.pad(emb1, ((0, 12), (0, 0)))
    wfa = Wf1[0:F]
    wv = Wf1[F:F + 2]
    wc = Wf1[F + 2:F + 34]
    wd = Wf1[F + 34:F + 66]

    degp = _deg_call(dstP, z1k)
    d0 = degp[0:N].reshape(N, 1)
    d1 = degp[NP:NP + N].reshape(N, 1)
    xsP = _xs_call(x, W1, d0, d1)
    accp = _agg1_call(srcP, dstP, xsP, zrF)
    g0, g1, dv, tab, parb = _mid_call(accp.reshape(2, NP, F),
                                      xsP.reshape(NP, F), d0, d1, b1, W2,
                                      wfa, wv, emb0p, emb1p,
                                      wc, wd, bf1, Wf2, bf2)
    g0P = g0.reshape(NP)
    g1P = g1.reshape(NP)
    accA, accB = _agg2_call(srcP, dstP, g0P, g1P, z1k)
    o0, o1 = _head_call(srcP, dstP, ea0P, ea1P, ea2P, ea3P,
                        accA, accB, g0P, g1P, dv.reshape(NP),
                        tab.reshape(256), parb.reshape(96))
    l0, l1 = _lsm_call(o0[0:ER], o1[0:ER])
    return jnp.concatenate(
        [l0.reshape(E, 1), l1.reshape(E, 1)], axis=1)
